# Initial kernel scaffold; baseline (speedup 1.0000x reference)
import jax
import jax.numpy as jnp
from jax.experimental import pallas as pl


def kernel(x, edge_index, batch_size, W1, b1, Wg, att_src, att_dst, bg, W2, b2, W3, b3):
    raise NotImplementedError("write your pallas kernel here")


# speedup vs baseline: 1.5370x; 1.5370x over previous
import functools

import jax
import jax.numpy as jnp
from jax import lax
from jax.experimental import pallas as pl

N = 10000
IN_DIM = 128
EMB = 64
HID = 64


def _prologue_body(x_ref, w1_ref, b1_ref, wg_ref, att2_ref, w2_ref, b2_ref,
                   w3_ref, b3_ref, hg_ref, a2_ref, xa_ref, m_ref):
    x = x_ref[...]
    h = jnp.maximum(
        lax.dot_general(x, w1_ref[...], (((1,), (1,)), ((), ())),
                        preferred_element_type=jnp.float32,
                        precision=lax.Precision.HIGHEST) + b1_ref[...][None, :],
        0.0)
    hg = lax.dot_general(h, wg_ref[...], (((1,), (1,)), ((), ())),
                         preferred_element_type=jnp.float32,
                         precision=lax.Precision.HIGHEST)
    hg_ref[...] = hg
    a2 = lax.dot_general(hg, att2_ref[...], (((1,), (0,)), ((), ())),
                         preferred_element_type=jnp.float32,
                         precision=lax.Precision.HIGHEST)
    a2_ref[...] = a2
    xt_w2 = lax.dot_general(x, w2_ref[...], (((0,), (1,)), ((), ())),
                            preferred_element_type=jnp.float32,
                            precision=lax.Precision.HIGHEST)
    xa1 = jnp.maximum(xt_w2 + b2_ref[...][None, :], 0.0)
    xa_ref[...] = lax.dot_general(xa1, w3_ref[...], (((1,), (1,)), ((), ())),
                                  preferred_element_type=jnp.float32,
                                  precision=lax.Precision.HIGHEST)
    t = jnp.max(a2[:, 0]) + jnp.max(a2[:, 1])
    m_ref[...] = jnp.where(t > 0, t, 0.2 * t).reshape(1, 1)


def _prologue(x, W1, b1, Wg, att2, W2, b2, W3, b3):
    return pl.pallas_call(
        _prologue_body,
        out_shape=(
            jax.ShapeDtypeStruct((N, HID), jnp.float32),
            jax.ShapeDtypeStruct((N, 2), jnp.float32),
            jax.ShapeDtypeStruct((IN_DIM, HID), jnp.float32),
            jax.ShapeDtypeStruct((1, 1), jnp.float32),
        ),
    )(x, W1, b1, Wg, att2, W2, b2, W3, b3)


BM = 1024
BN = 1024


def _smat_body(a_ref, b_ref, o_ref):
    s = lax.dot_general(a_ref[...], b_ref[...], (((1,), (1,)), ((), ())),
                        preferred_element_type=jnp.float32,
                        precision=lax.Precision.HIGHEST)
    o_ref[...] = jax.nn.sigmoid(s)


def _smat(emb):
    grid = (pl.cdiv(N, BM), pl.cdiv(N, BN))
    return pl.pallas_call(
        _smat_body,
        grid=grid,
        in_specs=[
            pl.BlockSpec((BM, HID), lambda i, j: (i, 0)),
            pl.BlockSpec((BN, HID), lambda i, j: (j, 0)),
        ],
        out_specs=pl.BlockSpec((BM, BN), lambda i, j: (i, j)),
        out_shape=jax.ShapeDtypeStruct((N, N), jnp.float32),
    )(emb, emb)


def _epilogue_body(numer_ref, denom_ref, bg_ref, xa_ref, emb_ref, xr_ref):
    emb = numer_ref[...] / (denom_ref[...] + 1e-16) + bg_ref[...][None, :]
    emb_ref[...] = emb
    xr_ref[...] = lax.dot_general(emb, xa_ref[...], (((1,), (1,)), ((), ())),
                                  preferred_element_type=jnp.float32,
                                  precision=lax.Precision.HIGHEST)


def _epilogue(numer, denom, bg, xa):
    return pl.pallas_call(
        _epilogue_body,
        out_shape=(
            jax.ShapeDtypeStruct((N, HID), jnp.float32),
            jax.ShapeDtypeStruct((N, IN_DIM), jnp.float32),
        ),
    )(numer, denom, bg, xa)


def kernel(x, edge_index, batch_size, W1, b1, Wg, att_src, att_dst, bg,
           W2, b2, W3, b3):
    att2 = jnp.stack([att_src, att_dst], axis=1)
    hg, a2, xa, M = _prologue(x, W1, b1, Wg, att2, W2, b2, W3, b3)
    a_src = a2[:, 0]
    a_dst = a2[:, 1]
    M = M[0, 0]

    loop = jnp.arange(N, dtype=edge_index.dtype)
    src = jnp.concatenate([edge_index[0], loop])
    dst = jnp.concatenate([edge_index[1], loop])
    t = a_src[src] + a_dst[dst]
    e = jnp.where(t > 0, t, 0.2 * t) - M
    w = jnp.exp(e)
    denom = jax.ops.segment_sum(w, dst, num_segments=N)
    numer = jax.ops.segment_sum(hg[src] * w[:, None], dst, num_segments=N)

    emb, x_ = _epilogue(numer, denom[:, None], bg, xa)
    s_ = _smat(emb)
    return (x_, s_)


# speedup vs baseline: 16.2454x; 10.5698x over previous
import functools

import jax
import jax.numpy as jnp
from jax import lax
from jax.experimental import pallas as pl
from jax.experimental.pallas import tpu as pltpu
from jax.experimental.pallas import tpu_sc as plsc

N = 10000
IN_DIM = 128
EMB = 64
HID = 64

NP = 10240
WIDTH = 80
NTILES = 32
CHUNK = 128
CH = 82
PAIRS = CH // 2
EPT = CH * CHUNK
EP = NTILES * EPT
ROWS_PT = NP // 16
COPY_ROWS = 80
NCOPY = ROWS_PT // COPY_ROWS


def _prologue_body(x_ref, w1_ref, b1_ref, wg_ref, att2_ref, w2_ref, b2_ref,
                   w3_ref, b3_ref, hg_ref, asrc_ref, adst_ref, m_ref, xa_ref):
    x = x_ref[...]
    h = jnp.maximum(
        lax.dot_general(x, w1_ref[...], (((1,), (1,)), ((), ())),
                        preferred_element_type=jnp.float32,
                        precision=lax.Precision.HIGHEST) + b1_ref[...][None, :],
        0.0)
    hg = lax.dot_general(h, wg_ref[...], (((1,), (1,)), ((), ())),
                         preferred_element_type=jnp.float32,
                         precision=lax.Precision.HIGHEST)
    hg_ref[0:N, :] = hg
    hg_ref[N:NP, :] = jnp.zeros((NP - N, EMB), jnp.float32)
    a2 = lax.dot_general(hg, att2_ref[...], (((1,), (0,)), ((), ())),
                         preferred_element_type=jnp.float32,
                         precision=lax.Precision.HIGHEST)
    asrc_ref[0:N] = a2[:, 0]
    asrc_ref[N:NP] = jnp.full((NP - N,), -1e9, jnp.float32)
    adst_ref[0:N] = a2[:, 1]
    adst_ref[N:NP] = jnp.full((NP - N,), -1e9, jnp.float32)
    xt_w2 = lax.dot_general(x, w2_ref[...], (((0,), (1,)), ((), ())),
                            preferred_element_type=jnp.float32,
                            precision=lax.Precision.HIGHEST)
    xa1 = jnp.maximum(xt_w2 + b2_ref[...][None, :], 0.0)
    xa_ref[...] = lax.dot_general(xa1, w3_ref[...], (((1,), (1,)), ((), ())),
                                  preferred_element_type=jnp.float32,
                                  precision=lax.Precision.HIGHEST)
    t = jnp.max(a2[:, 0]) + jnp.max(a2[:, 1])
    m_ref[...] = jnp.broadcast_to(jnp.where(t > 0, t, 0.2 * t), (16,))


def _prologue(x, W1, b1, Wg, att2, W2, b2, W3, b3):
    return pl.pallas_call(
        _prologue_body,
        out_shape=(
            jax.ShapeDtypeStruct((NP, EMB), jnp.float32),
            jax.ShapeDtypeStruct((NP,), jnp.float32),
            jax.ShapeDtypeStruct((NP,), jnp.float32),
            jax.ShapeDtypeStruct((16,), jnp.float32),
            jax.ShapeDtypeStruct((IN_DIM, HID), jnp.float32),
        ),
    )(x, W1, b1, Wg, att2, W2, b2, W3, b3)


def _edge_sc(src3, dst3, asrc, adst, m16, hg_ext):
    mesh = plsc.VectorSubcoreMesh(core_axis_name="c", subcore_axis_name="s")

    @functools.partial(
        pl.kernel,
        mesh=mesh,
        compiler_params=pltpu.CompilerParams(needs_layout_passes=False,
                                             use_tc_tiling_on_sc=False),
        out_type=jax.ShapeDtypeStruct((2, NP, WIDTH), jnp.float32),
        scratch_types=[
            pltpu.VMEM((CH, CHUNK), jnp.int32),
            pltpu.VMEM((CH, CHUNK), jnp.int32),
            pltpu.VMEM((NP,), jnp.float32),
            pltpu.VMEM((NP,), jnp.float32),
            pltpu.VMEM((16,), jnp.float32),
            pltpu.VMEM((CHUNK,), jnp.float32),
            pltpu.VMEM((CHUNK, EMB), jnp.float32),
            pltpu.VMEM((CHUNK, EMB), jnp.float32),
            pltpu.VMEM((CHUNK, WIDTH), jnp.float32),
            pltpu.VMEM((COPY_ROWS, WIDTH), jnp.float32),
            pltpu.VMEM_SHARED((NP, WIDTH), jnp.float32),
            pltpu.SemaphoreType.DMA,
            pltpu.SemaphoreType.DMA,
            pltpu.SemaphoreType.DMA,
        ],
    )
    def k(src_hbm, dst_hbm, asrc_hbm, adst_hbm, m_hbm, hg_hbm, out_hbm,
          src2, dst2, asrc_v, adst_v, m_v, w1d, gbuf0, gbuf1, sbuf, zbuf,
          acc, gsem0, gsem1, ssem):
        cidx = lax.axis_index("c")
        sidx = lax.axis_index("s")
        wid = cidx * 16 + sidx

        def zrow(r, _):
            for q in range(WIDTH // 16):
                zbuf[r, pl.ds(16 * q, 16)] = jnp.zeros((16,), jnp.float32)
            return 0
        lax.fori_loop(0, COPY_ROWS, zrow, 0)
        base_rows = sidx * ROWS_PT

        def zcopy(h, _):
            pltpu.sync_copy(
                zbuf, acc.at[pl.ds(base_rows + h * COPY_ROWS, COPY_ROWS)])
            return 0
        lax.fori_loop(0, NCOPY, zcopy, 0)

        pltpu.sync_copy(src_hbm.at[wid], src2)
        pltpu.sync_copy(dst_hbm.at[wid], dst2)
        pltpu.sync_copy(asrc_hbm, asrc_v)
        pltpu.sync_copy(adst_hbm, adst_v)
        pltpu.sync_copy(m_hbm, m_v)
        mv = m_v[...]

        plsc.subcore_barrier()

        lane0 = lax.iota(jnp.int32, 16) == 0
        zero16 = jnp.zeros((16,), jnp.float32)

        pltpu.async_copy(hg_hbm.at[src2.at[0]], gbuf0, gsem0)
        pltpu.async_copy(hg_hbm.at[src2.at[1]], gbuf1, gsem1)

        def pair(p, _):
            for b in (0, 1):
                c = 2 * p + b
                gb = gbuf0 if b == 0 else gbuf1
                gs = gsem0 if b == 0 else gsem1
                for q in range(CHUNK // 16):
                    s16 = src2[c, pl.ds(16 * q, 16)]
                    d16 = dst2[c, pl.ds(16 * q, 16)]
                    a_s = plsc.load_gather(asrc_v, [s16])
                    a_d = plsc.load_gather(adst_v, [d16])
                    t = a_s + a_d
                    e = jnp.where(t > 0.0, t, 0.2 * t) - mv
                    w1d[pl.ds(16 * q, 16)] = jnp.exp(e)
                pltpu.make_async_copy(hg_hbm.at[src2.at[c]], gb, gs).wait()
                if b == 0:
                    @pl.when(p > 0)
                    def _():
                        pltpu.make_async_copy(
                            sbuf, acc.at[dst2.at[c - 1]], ssem).wait()
                else:
                    pltpu.make_async_copy(
                        sbuf, acc.at[dst2.at[c - 1]], ssem).wait()

                def sgroup(g, _):
                    wrow = w1d[pl.ds(16 * g, 16)]
                    for j in range(16):
                        r = 16 * g + j
                        wv = lax.broadcast_in_dim(wrow[j], (16,), ())
                        for qq in range(EMB // 16):
                            sbuf[r, pl.ds(16 * qq, 16)] = (
                                gb[r, pl.ds(16 * qq, 16)] * wv)
                        sbuf[r, pl.ds(EMB, 16)] = jnp.where(lane0, wv, zero16)
                    return 0
                lax.fori_loop(0, CHUNK // 16, sgroup, 0)

                @pl.when(c + 2 < CH)
                def _():
                    pltpu.async_copy(hg_hbm.at[src2.at[c + 2]], gb, gs)
                pltpu.async_copy(sbuf, acc.at[dst2.at[c]], ssem, add=True)
            return 0
        lax.fori_loop(0, PAIRS, pair, 0)
        pltpu.make_async_copy(sbuf, acc.at[dst2.at[CH - 1]], ssem).wait()

        plsc.subcore_barrier()

        def ocopy(h, _):
            r0 = base_rows + h * COPY_ROWS
            pltpu.sync_copy(acc.at[pl.ds(r0, COPY_ROWS)], zbuf)
            pltpu.sync_copy(zbuf, out_hbm.at[cidx, pl.ds(r0, COPY_ROWS)])
            return 0
        lax.fori_loop(0, NCOPY, ocopy, 0)

    return k(src3, dst3, asrc, adst, m16, hg_ext)


def _epilogue_body(p0_ref, p1_ref, bg_ref, xa_ref, emb_ref, xr_ref):
    tot = p0_ref[...] + p1_ref[...]
    feat = lax.slice(tot, (0, 0), (N, EMB))
    den = lax.slice(tot, (0, EMB), (N, EMB + 1))
    emb = feat / (den + 1e-16) + bg_ref[...][None, :]
    emb_ref[...] = emb
    xr_ref[...] = lax.dot_general(emb, xa_ref[...], (((1,), (1,)), ((), ())),
                                  preferred_element_type=jnp.float32,
                                  precision=lax.Precision.HIGHEST)


def _epilogue(p0, p1, bg, xa):
    return pl.pallas_call(
        _epilogue_body,
        out_shape=(
            jax.ShapeDtypeStruct((N, HID), jnp.float32),
            jax.ShapeDtypeStruct((N, IN_DIM), jnp.float32),
        ),
    )(p0, p1, bg, xa)


BM = 2048
BN = 2048


def _smat_body(a_ref, b_ref, o_ref):
    s = lax.dot_general(a_ref[...], b_ref[...], (((1,), (1,)), ((), ())),
                        preferred_element_type=jnp.float32,
                        precision=lax.Precision.HIGHEST)
    o_ref[...] = jax.nn.sigmoid(s)


def _smat(emb):
    grid = (pl.cdiv(N, BM), pl.cdiv(N, BN))
    return pl.pallas_call(
        _smat_body,
        grid=grid,
        in_specs=[
            pl.BlockSpec((BM, HID), lambda i, j: (i, 0)),
            pl.BlockSpec((BN, HID), lambda i, j: (j, 0)),
        ],
        out_specs=pl.BlockSpec((BM, BN), lambda i, j: (i, j)),
        out_shape=jax.ShapeDtypeStruct((N, N), jnp.float32),
    )(emb, emb)


def kernel(x, edge_index, batch_size, W1, b1, Wg, att_src, att_dst, bg,
           W2, b2, W3, b3):
    att2 = jnp.stack([att_src, att_dst], axis=1)
    hg_ext, asrc, adst, m16, xa = _prologue(x, W1, b1, Wg, att2, W2, b2,
                                            W3, b3)

    loop = jnp.arange(N, dtype=jnp.int32)
    npad = EP - (edge_index.shape[1] + N)
    pad = N + (jnp.arange(npad, dtype=jnp.int32) % (NP - N))
    src3 = jnp.concatenate([edge_index[0].astype(jnp.int32), loop, pad]
                           ).reshape(NTILES, CH, CHUNK)
    dst3 = jnp.concatenate([edge_index[1].astype(jnp.int32), loop, pad]
                           ).reshape(NTILES, CH, CHUNK)

    partials = _edge_sc(src3, dst3, asrc, adst, m16, hg_ext)
    emb, x_ = _epilogue(partials[0], partials[1], bg, xa)
    s_ = _smat(emb)
    return (x_, s_)


# speedup vs baseline: 23.9467x; 1.4741x over previous
import functools

import jax
import jax.numpy as jnp
from jax import lax
from jax.experimental import pallas as pl
from jax.experimental.pallas import tpu as pltpu
from jax.experimental.pallas import tpu_sc as plsc

N = 10000
IN_DIM = 128
EMB = 64
HID = 64

NP = 10240
WIDTH = 80
NTILES = 32
CHUNK = 128
CH = 82
PAIRS = CH // 2
EPT = CH * CHUNK
EP = NTILES * EPT
ROWS_PT = NP // 16
COPY_ROWS = 80
NCOPY = ROWS_PT // COPY_ROWS


def _prologue_body(x_ref, w1_ref, b1_ref, wg_ref, att2_ref, w2_ref, b2_ref,
                   w3_ref, b3_ref, hg_ref, asrc_ref, adst_ref, m_ref, xa_ref):
    x = x_ref[...]
    h = jnp.maximum(
        lax.dot_general(x, w1_ref[...], (((1,), (1,)), ((), ())),
                        preferred_element_type=jnp.float32,
                        precision=lax.Precision.HIGHEST) + b1_ref[...][None, :],
        0.0)
    hg = lax.dot_general(h, wg_ref[...], (((1,), (1,)), ((), ())),
                         preferred_element_type=jnp.float32,
                         precision=lax.Precision.HIGHEST)
    hg_ref[0:N, :] = hg
    hg_ref[N:NP, :] = jnp.zeros((NP - N, EMB), jnp.float32)
    a2 = lax.dot_general(hg, att2_ref[...], (((1,), (0,)), ((), ())),
                         preferred_element_type=jnp.float32,
                         precision=lax.Precision.HIGHEST)
    asrc_ref[0:N] = a2[:, 0]
    asrc_ref[N:NP] = jnp.full((NP - N,), -1e9, jnp.float32)
    adst_ref[0:N] = a2[:, 1]
    adst_ref[N:NP] = jnp.full((NP - N,), -1e9, jnp.float32)
    xt_w2 = lax.dot_general(x, w2_ref[...], (((0,), (1,)), ((), ())),
                            preferred_element_type=jnp.float32,
                            precision=lax.Precision.HIGHEST)
    xa1 = jnp.maximum(xt_w2 + b2_ref[...][None, :], 0.0)
    xa_ref[...] = lax.dot_general(xa1, w3_ref[...], (((1,), (1,)), ((), ())),
                                  preferred_element_type=jnp.float32,
                                  precision=lax.Precision.HIGHEST)
    t = jnp.max(a2[:, 0]) + jnp.max(a2[:, 1])
    m_ref[...] = jnp.broadcast_to(jnp.where(t > 0, t, 0.2 * t), (16,))


def _prologue(x, W1, b1, Wg, att2, W2, b2, W3, b3):
    return pl.pallas_call(
        _prologue_body,
        out_shape=(
            jax.ShapeDtypeStruct((NP, EMB), jnp.float32),
            jax.ShapeDtypeStruct((NP,), jnp.float32),
            jax.ShapeDtypeStruct((NP,), jnp.float32),
            jax.ShapeDtypeStruct((16,), jnp.float32),
            jax.ShapeDtypeStruct((IN_DIM, HID), jnp.float32),
        ),
    )(x, W1, b1, Wg, att2, W2, b2, W3, b3)


def _edge_sc(src3, dst3, asrc, adst, m16, hg_ext):
    mesh = plsc.VectorSubcoreMesh(core_axis_name="c", subcore_axis_name="s")

    @functools.partial(
        pl.kernel,
        mesh=mesh,
        compiler_params=pltpu.CompilerParams(needs_layout_passes=False,
                                             use_tc_tiling_on_sc=False),
        out_type=jax.ShapeDtypeStruct((2, NP, WIDTH), jnp.float32),
        scratch_types=[
            pltpu.VMEM((CH, CHUNK), jnp.int32),
            pltpu.VMEM((CH, CHUNK), jnp.int32),
            pltpu.VMEM((NP,), jnp.float32),
            pltpu.VMEM((NP,), jnp.float32),
            pltpu.VMEM((16,), jnp.float32),
            pltpu.VMEM((CHUNK,), jnp.float32),
            pltpu.VMEM((CHUNK, EMB), jnp.float32),
            pltpu.VMEM((CHUNK, EMB), jnp.float32),
            pltpu.VMEM((CHUNK, WIDTH), jnp.float32),
            pltpu.VMEM((COPY_ROWS, WIDTH), jnp.float32),
            pltpu.VMEM_SHARED((NP, WIDTH), jnp.float32),
            pltpu.SemaphoreType.DMA,
            pltpu.SemaphoreType.DMA,
            pltpu.SemaphoreType.DMA,
        ],
    )
    def k(src_hbm, dst_hbm, asrc_hbm, adst_hbm, m_hbm, hg_hbm, out_hbm,
          src2, dst2, asrc_v, adst_v, m_v, w1d, gbuf0, gbuf1, sbuf, zbuf,
          acc, gsem0, gsem1, ssem):
        cidx = lax.axis_index("c")
        sidx = lax.axis_index("s")
        wid = cidx * 16 + sidx

        def zrow(r, _):
            for q in range(WIDTH // 16):
                zbuf[r, pl.ds(16 * q, 16)] = jnp.zeros((16,), jnp.float32)
            return 0
        lax.fori_loop(0, COPY_ROWS, zrow, 0)
        base_rows = sidx * ROWS_PT

        def zcopy(h, _):
            pltpu.sync_copy(
                zbuf, acc.at[pl.ds(base_rows + h * COPY_ROWS, COPY_ROWS)])
            return 0
        lax.fori_loop(0, NCOPY, zcopy, 0)

        pltpu.sync_copy(src_hbm.at[wid], src2)
        pltpu.sync_copy(dst_hbm.at[wid], dst2)
        pltpu.sync_copy(asrc_hbm, asrc_v)
        pltpu.sync_copy(adst_hbm, adst_v)
        pltpu.sync_copy(m_hbm, m_v)
        mv = m_v[...]

        plsc.subcore_barrier()

        lane0 = lax.iota(jnp.int32, 16) == 0
        zero16 = jnp.zeros((16,), jnp.float32)

        pltpu.async_copy(hg_hbm.at[src2.at[0]], gbuf0, gsem0)
        pltpu.async_copy(hg_hbm.at[src2.at[1]], gbuf1, gsem1)

        def pair(p, _):
            for b in (0, 1):
                c = 2 * p + b
                gb = gbuf0 if b == 0 else gbuf1
                gs = gsem0 if b == 0 else gsem1
                for q in range(CHUNK // 16):
                    s16 = src2[c, pl.ds(16 * q, 16)]
                    d16 = dst2[c, pl.ds(16 * q, 16)]
                    a_s = plsc.load_gather(asrc_v, [s16])
                    a_d = plsc.load_gather(adst_v, [d16])
                    t = a_s + a_d
                    e = jnp.where(t > 0.0, t, 0.2 * t) - mv
                    w1d[pl.ds(16 * q, 16)] = jnp.exp(e)
                pltpu.make_async_copy(hg_hbm.at[src2.at[c]], gb, gs).wait()
                if b == 0:
                    @pl.when(p > 0)
                    def _():
                        pltpu.make_async_copy(
                            sbuf, acc.at[dst2.at[c - 1]], ssem).wait()
                else:
                    pltpu.make_async_copy(
                        sbuf, acc.at[dst2.at[c - 1]], ssem).wait()

                def sgroup(g, _):
                    wrow = w1d[pl.ds(16 * g, 16)]
                    for j in range(16):
                        r = 16 * g + j
                        wv = lax.broadcast_in_dim(wrow[j], (16,), ())
                        for qq in range(EMB // 16):
                            sbuf[r, pl.ds(16 * qq, 16)] = (
                                gb[r, pl.ds(16 * qq, 16)] * wv)
                        sbuf[r, pl.ds(EMB, 16)] = jnp.where(lane0, wv, zero16)
                    return 0
                lax.fori_loop(0, CHUNK // 16, sgroup, 0)

                @pl.when(c + 2 < CH)
                def _():
                    pltpu.async_copy(hg_hbm.at[src2.at[c + 2]], gb, gs)
                pltpu.async_copy(sbuf, acc.at[dst2.at[c]], ssem, add=True)
            return 0
        lax.fori_loop(0, PAIRS, pair, 0)
        pltpu.make_async_copy(sbuf, acc.at[dst2.at[CH - 1]], ssem).wait()

        plsc.subcore_barrier()

        def ocopy(h, _):
            r0 = base_rows + h * COPY_ROWS
            pltpu.sync_copy(acc.at[pl.ds(r0, COPY_ROWS)], zbuf)
            pltpu.sync_copy(zbuf, out_hbm.at[cidx, pl.ds(r0, COPY_ROWS)])
            return 0
        lax.fori_loop(0, NCOPY, ocopy, 0)

    return k(src3, dst3, asrc, adst, m16, hg_ext)


def _epilogue_body(p0_ref, p1_ref, bg_ref, xa_ref, emb_ref, xr_ref):
    tot = p0_ref[...] + p1_ref[...]
    feat = lax.slice(tot, (0, 0), (N, EMB))
    den = lax.slice(tot, (0, EMB), (N, EMB + 1))
    emb = feat / (den + 1e-16) + bg_ref[...][None, :]
    emb_ref[...] = emb
    xr_ref[...] = lax.dot_general(emb, xa_ref[...], (((1,), (1,)), ((), ())),
                                  preferred_element_type=jnp.float32,
                                  precision=lax.Precision.HIGHEST)


def _epilogue(p0, p1, bg, xa):
    return pl.pallas_call(
        _epilogue_body,
        out_shape=(
            jax.ShapeDtypeStruct((N, HID), jnp.float32),
            jax.ShapeDtypeStruct((N, IN_DIM), jnp.float32),
        ),
    )(p0, p1, bg, xa)


BM = 2048
BN = 2048


def _smat_body(a_ref, b_ref, o_ref):
    s = lax.dot_general(a_ref[...], b_ref[...], (((1,), (1,)), ((), ())),
                        preferred_element_type=jnp.float32)
    o_ref[...] = jax.nn.sigmoid(s)


def _smat(emb):
    grid = (pl.cdiv(N, BM), pl.cdiv(N, BN))
    return pl.pallas_call(
        _smat_body,
        grid=grid,
        in_specs=[
            pl.BlockSpec((BM, HID), lambda i, j: (i, 0)),
            pl.BlockSpec((BN, HID), lambda i, j: (j, 0)),
        ],
        out_specs=pl.BlockSpec((BM, BN), lambda i, j: (i, j)),
        out_shape=jax.ShapeDtypeStruct((N, N), jnp.float32),
    )(emb, emb)


def kernel(x, edge_index, batch_size, W1, b1, Wg, att_src, att_dst, bg,
           W2, b2, W3, b3):
    att2 = jnp.stack([att_src, att_dst], axis=1)
    hg_ext, asrc, adst, m16, xa = _prologue(x, W1, b1, Wg, att2, W2, b2,
                                            W3, b3)

    loop = jnp.arange(N, dtype=jnp.int32)
    npad = EP - (edge_index.shape[1] + N)
    pad = N + (jnp.arange(npad, dtype=jnp.int32) % (NP - N))
    src3 = jnp.concatenate([edge_index[0].astype(jnp.int32), loop, pad]
                           ).reshape(NTILES, CH, CHUNK)
    dst3 = jnp.concatenate([edge_index[1].astype(jnp.int32), loop, pad]
                           ).reshape(NTILES, CH, CHUNK)

    partials = _edge_sc(src3, dst3, asrc, adst, m16, hg_ext)
    emb, x_ = _epilogue(partials[0], partials[1], bg, xa)
    s_ = _smat(emb)
    return (x_, s_)
